# baseline (speedup 1.0000x reference)
import functools

import jax
import jax.numpy as jnp
from jax import lax
from jax.experimental import pallas as pl
from jax.experimental.pallas import tpu as pltpu
from jax.experimental.pallas import tpu_sc as plsc

BATCH = 4096
HIST = 20
DIM = 64
NUM_ROWS = BATCH * HIST
CHUNK = 128
N_CHUNKS = NUM_ROWS // CHUNK
NC, NS = 2, 16
NW = NC * NS
CHUNKS_PER_W = N_CHUNKS // NW
WAVE = CHUNKS_PER_W // 2

_mesh = plsc.VectorSubcoreMesh(core_axis_name="c", subcore_axis_name="s")


@functools.partial(
    pl.kernel,
    mesh=_mesh,
    out_type=jax.ShapeDtypeStruct((N_CHUNKS, CHUNK, DIM), jnp.float32),
    scratch_types=[
        pltpu.VMEM((CHUNKS_PER_W, CHUNK), jnp.int32),
        pltpu.VMEM((WAVE, CHUNK, DIM), jnp.float32),
        pltpu.SemaphoreType.DMA,
    ],
    compiler_params=pltpu.CompilerParams(use_tc_tiling_on_sc=False),
)
def _embed_gather(idx_hbm, table_hbm, out_hbm, idx_v, rows_v, sem):
    wid = lax.axis_index("s") * NC + lax.axis_index("c")
    base = wid * CHUNKS_PER_W
    pltpu.sync_copy(idx_hbm.at[wid], idx_v)
    for p in range(CHUNKS_PER_W // WAVE):
        copies = [
            pltpu.async_copy(
                table_hbm.at[idx_v.at[p * WAVE + j]], rows_v.at[j], sem
            )
            for j in range(WAVE)
        ]
        for c in copies:
            c.wait()
        pltpu.sync_copy(rows_v, out_hbm.at[pl.ds(base + p * WAVE, WAVE)])


def kernel(x, table):
    idx = x.reshape(NW, CHUNKS_PER_W, CHUNK).astype(jnp.int32)
    out = _embed_gather(idx, table)
    return out.reshape(BATCH, HIST, DIM)
